# baseline (speedup 1.0000x reference)
import functools
import math

import jax
import jax.numpy as jnp
from jax import lax
from jax.experimental import pallas as pl
from jax.experimental.pallas import tpu as pltpu
from jax.experimental.pallas import tpu_sc as plsc

DIM = 64
SCALE = math.sqrt(DIM)
NW = 32
CHUNK = 128
NCH = 50
LANES = 16
ROWS_PER_IT = 4


def _body(idx_hbm, table_hbm, out_hbm, idx_v, b0, b1, b2, g0, g1, g2,
          o0, o1, o2):
    bufs = (b0, b1, b2)
    gsems = (g0, g1, g2)
    osems = (o0, o1, o2)
    wid = lax.axis_index("s") * 2 + lax.axis_index("c")

    pltpu.sync_copy(idx_hbm.at[wid], idx_v)

    def gather_start(c, b):
        pltpu.async_copy(table_hbm.at[idx_v.at[c]], bufs[b], gsems[b])

    def gather_wait(c, b):
        pltpu.make_async_copy(table_hbm.at[idx_v.at[c]], bufs[b],
                              gsems[b]).wait()

    def out_start(c, b):
        pltpu.async_copy(bufs[b], out_hbm.at[wid, c], osems[b])

    def out_wait(c, b):
        pltpu.make_async_copy(bufs[b], out_hbm.at[wid, c], osems[b]).wait()

    def scale(b):
        buf = bufs[b]

        def row_block(i, carry):
            for u in range(ROWS_PER_IT):
                r = i * ROWS_PER_IT + u
                for k in range(DIM // LANES):
                    sl = pl.ds(k * LANES, LANES)
                    buf[r, sl] = buf[r, sl] * SCALE
            return carry

        lax.fori_loop(0, CHUNK // ROWS_PER_IT, row_block, 0)

    gather_start(0, 0)
    gather_start(1, 1)

    gather_wait(0, 0)
    scale(0)
    out_start(0, 0)
    gather_start(2, 2)

    gather_wait(1, 1)
    scale(1)
    out_start(1, 1)
    out_wait(0, 0)
    gather_start(3, 0)

    def trio(t, carry):
        j = 2 + 3 * t
        for b3 in range(3):
            c = j + b3
            bb = (2 + b3) % 3
            nb = (bb + 2) % 3
            gather_wait(c, bb)
            scale(bb)
            out_start(c, bb)

            @pl.when(c < NCH - 2)
            def _():
                out_wait(c - 1, nb)
                gather_start(c + 2, nb)

        return carry

    lax.fori_loop(0, (NCH - 2) // 3, trio, 0)

    out_wait(NCH - 3, (NCH - 3) % 3)
    out_wait(NCH - 2, (NCH - 2) % 3)
    out_wait(NCH - 1, (NCH - 1) % 3)


@functools.partial(
    pl.kernel,
    out_type=jax.ShapeDtypeStruct((NW, NCH, CHUNK, DIM), jnp.float32),
    mesh=plsc.VectorSubcoreMesh(core_axis_name="c", subcore_axis_name="s"),
    compiler_params=pltpu.CompilerParams(use_tc_tiling_on_sc=False),
    scratch_types=[
        pltpu.VMEM((NCH, CHUNK), jnp.int32),
        pltpu.VMEM((CHUNK, DIM), jnp.float32),
        pltpu.VMEM((CHUNK, DIM), jnp.float32),
        pltpu.VMEM((CHUNK, DIM), jnp.float32),
        pltpu.SemaphoreType.DMA,
        pltpu.SemaphoreType.DMA,
        pltpu.SemaphoreType.DMA,
        pltpu.SemaphoreType.DMA,
        pltpu.SemaphoreType.DMA,
        pltpu.SemaphoreType.DMA,
    ],
)
def _emb_lookup(idx_hbm, table_hbm, out_hbm, *rest):
    _body(idx_hbm, table_hbm, out_hbm, *rest)


def kernel(input_vec, table):
    b, s = input_vec.shape
    idx = input_vec.astype(jnp.int32).reshape(NW, NCH, CHUNK)
    out = _emb_lookup(idx, table)
    return out.reshape(b, s, DIM)
